# Initial kernel scaffold; baseline (speedup 1.0000x reference)
import jax
import jax.numpy as jnp
from jax.experimental import pallas as pl


def kernel(x, target):
    raise NotImplementedError("write your pallas kernel here")


# speedup vs baseline: 7.7532x; 7.7532x over previous
import math

import jax
import jax.numpy as jnp
import numpy as np
from jax.experimental import pallas as pl

_PAD = 0
_SMOOTHING = 0.1
_CONF = 1.0 - _SMOOTHING


def _row_block_body(smooth, K, x_ref, t_ref, out_ref):
    i = pl.program_id(0)
    xb = x_ref[...]
    tb = t_ref[0, 0, :]
    rowsum = jnp.sum(xb, axis=1)
    col0 = xb[:, 0]
    cols = jax.lax.broadcasted_iota(jnp.int32, xb.shape, 1)
    g = jnp.sum(jnp.where(cols == tb[:, None], xb, 0.0), axis=1)
    valid = tb != _PAD
    contrib = jnp.where(valid, K - smooth * (rowsum - col0) + (smooth - _CONF) * g, 0.0)
    s = jnp.sum(contrib).reshape(1, 1)

    @pl.when(i == 0)
    def _():
        out_ref[...] = jnp.zeros_like(out_ref)

    out_ref[...] += s


def kernel(x, target):
    n, size = x.shape
    smooth = float(np.float32(_SMOOTHING / (size - 2)))
    K = (size - 2) * smooth * math.log(smooth) + _CONF * math.log(_CONF)

    BR = 64
    nb = n // BR
    t3 = target.reshape(nb, 1, BR)

    out = pl.pallas_call(
        lambda x_ref, t_ref, o_ref: _row_block_body(
            jnp.float32(smooth), jnp.float32(K), x_ref, t_ref, o_ref),
        grid=(nb,),
        in_specs=[
            pl.BlockSpec((BR, size), lambda i: (i, 0)),
            pl.BlockSpec((1, 1, BR), lambda i: (i, 0, 0)),
        ],
        out_specs=pl.BlockSpec((1, 1), lambda i: (0, 0)),
        out_shape=jax.ShapeDtypeStruct((1, 1), jnp.float32),
    )(x, t3)
    return out[0, 0]
